# Initial kernel scaffold; baseline (speedup 1.0000x reference)
import jax
import jax.numpy as jnp
from jax.experimental import pallas as pl


def kernel(embeddings, user, item_p, item_n, edge_index):
    raise NotImplementedError("write your pallas kernel here")


# speedup vs baseline: 2.9925x; 2.9925x over previous
import jax
import jax.numpy as jnp
from jax import lax
from jax.experimental import pallas as pl
from jax.experimental.pallas import tpu as pltpu, tpu_sc as plsc

NU = 50000
NN = 100000
DD = 32
EE = 1600000
BB = 4096

NC, NS = 2, 16
HALF = NN // NC
ACC_ROWS = 50176
ZB_ROWS = 392
CH = 80
EPT = EE // NS
NCHUNK = EPT // CH
WB = 3128
WB_LAST = HALF - (NS - 1) * WB

_MESH = dict(core_axis_name="c", subcore_axis_name="s")


def _zero_fill(zbuf, width):
    zv = jnp.zeros((16,), jnp.float32)
    ngroups = width // 16

    def zb(i, carry):
        r = i // ngroups
        g = i % ngroups
        zbuf[r, pl.ds(g * 16, 16)] = zv
        return carry

    lax.fori_loop(0, ZB_ROWS * ngroups, zb, 0)


def _localize_dst(didx, base):
    for g in range(CH // 16):
        d = didx[pl.ds(g * 16, 16)]
        loc = d - base
        ok = (loc >= 0) & (loc < HALF)
        didx[pl.ds(g * 16, 16)] = jnp.where(ok, loc, HALF)


def _prop_body(feat_hbm, src_hbm, dst_hbm, out_hbm, acc, zbuf, rows_v, sidx, didx, sem):
    c = lax.axis_index("c")
    s = lax.axis_index("s")
    base = c * HALF

    _zero_fill(zbuf, DD)
    for b in range(8):
        pltpu.sync_copy(zbuf, acc.at[pl.ds(pl.multiple_of(s * 3136 + b * ZB_ROWS, 8), ZB_ROWS)])
    plsc.subcore_barrier()

    e0 = s * EPT

    def step(i, carry):
        eoff = pl.multiple_of(e0 + i * CH, 8)
        pltpu.sync_copy(src_hbm.at[pl.ds(eoff, CH)], sidx)
        pltpu.sync_copy(dst_hbm.at[pl.ds(eoff, CH)], didx)
        _localize_dst(didx, base)
        pltpu.async_copy(feat_hbm.at[sidx], rows_v, sem).wait()
        pltpu.sync_copy(rows_v, acc.at[didx], add=True)
        return carry

    lax.fori_loop(0, NCHUNK, step, 0)
    plsc.subcore_barrier()
    _writeback(acc, out_hbm, s, base)


def _writeback(acc, out_hbm, s, base):
    @pl.when(s < NS - 1)
    def _():
        off = pl.multiple_of(s * WB, 8)
        pltpu.sync_copy(acc.at[pl.ds(off, WB)],
                        out_hbm.at[pl.ds(pl.multiple_of(base + off, 8), WB)])

    @pl.when(s == NS - 1)
    def _():
        off = (NS - 1) * WB
        pltpu.sync_copy(acc.at[pl.ds(off, WB_LAST)],
                        out_hbm.at[pl.ds(pl.multiple_of(base + off, 8), WB_LAST)])


_prop = pl.kernel(
    _prop_body,
    out_type=jax.ShapeDtypeStruct((NN, DD), jnp.float32),
    mesh=plsc.VectorSubcoreMesh(num_cores=NC, num_subcores=NS, **_MESH),
    compiler_params=pltpu.CompilerParams(use_tc_tiling_on_sc=False),
    scratch_types=[
        pltpu.VMEM_SHARED((ACC_ROWS, DD), jnp.float32),
        pltpu.VMEM((ZB_ROWS, DD), jnp.float32),
        pltpu.VMEM((CH, DD), jnp.float32),
        pltpu.VMEM((CH,), jnp.int32),
        pltpu.VMEM((CH,), jnp.int32),
        pltpu.SemaphoreType.DMA,
    ],
)


def _deg_body(dst_hbm, out_hbm, acc, zbuf, ones_v, didx):
    c = lax.axis_index("c")
    s = lax.axis_index("s")
    base = c * HALF

    _zero_fill(zbuf, DD)
    for b in range(8):
        pltpu.sync_copy(zbuf, acc.at[pl.ds(pl.multiple_of(s * 3136 + b * ZB_ROWS, 8), ZB_ROWS)])
    ov = jnp.ones((16,), jnp.float32)

    def ob(i, carry):
        r = i // (DD // 16)
        g = i % (DD // 16)
        ones_v[r, pl.ds(g * 16, 16)] = ov
        return carry

    lax.fori_loop(0, CH * (DD // 16), ob, 0)
    plsc.subcore_barrier()

    e0 = s * EPT

    def step(i, carry):
        pltpu.sync_copy(dst_hbm.at[pl.ds(pl.multiple_of(e0 + i * CH, 8), CH)], didx)
        _localize_dst(didx, base)
        pltpu.sync_copy(ones_v, acc.at[didx], add=True)
        return carry

    lax.fori_loop(0, NCHUNK, step, 0)
    plsc.subcore_barrier()
    _writeback(acc, out_hbm, s, base)


_deg = pl.kernel(
    _deg_body,
    out_type=jax.ShapeDtypeStruct((NN, DD), jnp.float32),
    mesh=plsc.VectorSubcoreMesh(num_cores=NC, num_subcores=NS, **_MESH),
    compiler_params=pltpu.CompilerParams(use_tc_tiling_on_sc=False),
    scratch_types=[
        pltpu.VMEM_SHARED((ACC_ROWS, DD), jnp.float32),
        pltpu.VMEM((ZB_ROWS, DD), jnp.float32),
        pltpu.VMEM((CH, DD), jnp.float32),
        pltpu.VMEM((CH,), jnp.int32),
    ],
)


def _gather3_body(feat_hbm, u_hbm, p_hbm, n_hbm, uo_hbm, po_hbm, no_hbm,
                  idxv, rows_v, sem):
    w = lax.axis_index("s") * NC + lax.axis_index("c")
    b0 = pl.multiple_of(w * (BB // (NC * NS)), 8)
    for ih, oh in ((u_hbm, uo_hbm), (p_hbm, po_hbm), (n_hbm, no_hbm)):
        pltpu.sync_copy(ih.at[pl.ds(b0, BB // (NC * NS))], idxv)
        pltpu.async_copy(feat_hbm.at[idxv], rows_v, sem).wait()
        pltpu.sync_copy(rows_v, oh.at[pl.ds(b0, BB // (NC * NS))])


_gather3 = pl.kernel(
    _gather3_body,
    out_type=(
        jax.ShapeDtypeStruct((BB, DD), jnp.float32),
        jax.ShapeDtypeStruct((BB, DD), jnp.float32),
        jax.ShapeDtypeStruct((BB, DD), jnp.float32),
    ),
    mesh=plsc.VectorSubcoreMesh(num_cores=NC, num_subcores=NS, **_MESH),
    compiler_params=pltpu.CompilerParams(use_tc_tiling_on_sc=False),
    scratch_types=[
        pltpu.VMEM((BB // (NC * NS),), jnp.int32),
        pltpu.VMEM((BB // (NC * NS), DD), jnp.float32),
        pltpu.SemaphoreType.DMA,
    ],
)


_ROWS = NN * DD // 128
_BLK = 1000
_GRID = _ROWS // _BLK
_spec = pl.BlockSpec((_BLK, 128), lambda i: (i, 0))
_fullshape = jax.ShapeDtypeStruct((_ROWS, 128), jnp.float32)


def _tc1_body(deg_ref, emb_ref, norm_ref, s0_ref):
    n = lax.rsqrt(jnp.maximum(deg_ref[...], 1.0))
    norm_ref[...] = n
    s0_ref[...] = emb_ref[...] * n


_tc1 = pl.pallas_call(
    _tc1_body,
    grid=(_GRID,),
    in_specs=[_spec, _spec],
    out_specs=[_spec, _spec],
    out_shape=[_fullshape, _fullshape],
)


def _tc2_body(a1_ref, norm_ref, h1_ref, s1_ref):
    n = norm_ref[...]
    h1 = a1_ref[...] * n
    h1_ref[...] = h1
    s1_ref[...] = h1 * n


_tc2 = pl.pallas_call(
    _tc2_body,
    grid=(_GRID,),
    in_specs=[_spec, _spec],
    out_specs=[_spec, _spec],
    out_shape=[_fullshape, _fullshape],
)


def _tc3_body(emb_ref, h1_ref, a2_ref, norm_ref, f_ref):
    f_ref[...] = (emb_ref[...] + h1_ref[...] + a2_ref[...] * norm_ref[...]) * (1.0 / 3.0)


_tc3 = pl.pallas_call(
    _tc3_body,
    grid=(_GRID,),
    in_specs=[_spec, _spec, _spec, _spec],
    out_specs=_spec,
    out_shape=_fullshape,
)


def _tc4_body(u_ref, p_ref, n_ref, ps_ref, ns_ref):
    u = u_ref[...]
    ps_ref[...] = jnp.sum(u * p_ref[...], axis=1, keepdims=True)
    ns_ref[...] = jnp.sum(u * n_ref[...], axis=1, keepdims=True)


_tc4 = pl.pallas_call(
    _tc4_body,
    out_shape=[
        jax.ShapeDtypeStruct((BB, 1), jnp.float32),
        jax.ShapeDtypeStruct((BB, 1), jnp.float32),
    ],
)


def kernel(embeddings, user, item_p, item_n, edge_index):
    src = edge_index[0]
    dst = edge_index[1]
    u_idx = user[:, 0]
    p_idx = item_p[:, 0] + NU
    n_idx = item_n[:, 0] + NU

    deg = _deg(dst)
    norm, s0 = _tc1(deg.reshape(_ROWS, 128), embeddings.reshape(_ROWS, 128))
    a1 = _prop(s0.reshape(NN, DD), src, dst)
    h1, s1 = _tc2(a1.reshape(_ROWS, 128), norm)
    a2 = _prop(s1.reshape(NN, DD), src, dst)
    feats = _tc3(embeddings.reshape(_ROWS, 128), h1, a2.reshape(_ROWS, 128), norm)
    ur, pr, nr = _gather3(feats.reshape(NN, DD), u_idx, p_idx, n_idx)
    p_score, n_score = _tc4(ur, pr, nr)
    return p_score, n_score


# speedup vs baseline: 5.2871x; 1.7668x over previous
import jax
import jax.numpy as jnp
from jax import lax
from jax.experimental import pallas as pl
from jax.experimental.pallas import tpu as pltpu, tpu_sc as plsc

NU = 50000
NN = 100000
DD = 32
EE = 1600000
BB = 4096

NC, NS = 2, 16
HALF = NN // NC
ACC_ROWS = 50048
ZB_ROWS = 391
CH = 80
KK = 5
SUP = CH * KK
EPT = EE // NS
NSUP = EPT // SUP
WB = 3128
WB_LAST = HALF - (NS - 1) * WB

_MESH = dict(core_axis_name="c", subcore_axis_name="s")


def _zero_fill(zbuf, width):
    zv = jnp.zeros((16,), jnp.float32)
    ngroups = width // 16

    def zb(i, carry):
        r = i // ngroups
        g = i % ngroups
        zbuf[r, pl.ds(g * 16, 16)] = zv
        return carry

    lax.fori_loop(0, ZB_ROWS * ngroups, zb, 0)


def _localize_dst(dstag, didx, j, base):
    for g in range(CH // 16):
        d = dstag[pl.ds(j * CH + g * 16, 16)]
        loc = d - base
        ok = (loc >= 0) & (loc < HALF)
        didx[pl.ds(g * 16, 16)] = jnp.where(ok, loc, HALF)


def _prop_body(feat_hbm, src_hbm, dst_hbm, out_hbm, *sc):
    acc, zbuf, sstag, dstag = sc[0], sc[1], sc[2], sc[3]
    rows = sc[4:4 + KK]
    didx = sc[4 + KK:4 + 2 * KK]
    sem = sc[4 + 2 * KK]
    c = lax.axis_index("c")
    s = lax.axis_index("s")
    base = c * HALF

    _zero_fill(zbuf, DD)
    for b in range(8):
        pltpu.sync_copy(zbuf, acc.at[pl.ds(s * 3128 + b * ZB_ROWS, ZB_ROWS)])
    plsc.subcore_barrier()

    e0 = s * EPT

    def step(i, carry):
        eoff = pl.multiple_of(e0 + i * SUP, 8)
        pltpu.sync_copy(src_hbm.at[pl.ds(eoff, SUP)], sstag)
        pltpu.sync_copy(dst_hbm.at[pl.ds(eoff, SUP)], dstag)
        descs = []
        for j in range(KK):
            descs.append(pltpu.async_copy(
                feat_hbm.at[sstag.at[pl.ds(j * CH, CH)]], rows[j], sem))
        for j in range(KK):
            _localize_dst(dstag, didx[j], j, base)
        for j in range(KK):
            descs[j].wait()
            pltpu.sync_copy(rows[j], acc.at[didx[j]], add=True)
        return carry

    lax.fori_loop(0, NSUP, step, 0)
    plsc.subcore_barrier()
    _writeback(acc, out_hbm, s, base)


def _writeback(acc, out_hbm, s, base):
    @pl.when(s < NS - 1)
    def _():
        off = pl.multiple_of(s * WB, 8)
        pltpu.sync_copy(acc.at[pl.ds(off, WB)],
                        out_hbm.at[pl.ds(pl.multiple_of(base + off, 8), WB)])

    @pl.when(s == NS - 1)
    def _():
        off = (NS - 1) * WB
        pltpu.sync_copy(acc.at[pl.ds(off, WB_LAST)],
                        out_hbm.at[pl.ds(pl.multiple_of(base + off, 8), WB_LAST)])


_prop = pl.kernel(
    _prop_body,
    out_type=jax.ShapeDtypeStruct((NN, DD), jnp.float32),
    mesh=plsc.VectorSubcoreMesh(num_cores=NC, num_subcores=NS, **_MESH),
    compiler_params=pltpu.CompilerParams(use_tc_tiling_on_sc=False),
    scratch_types=(
        [
            pltpu.VMEM_SHARED((ACC_ROWS, DD), jnp.float32),
            pltpu.VMEM((ZB_ROWS, DD), jnp.float32),
            pltpu.VMEM((SUP,), jnp.int32),
            pltpu.VMEM((SUP,), jnp.int32),
        ]
        + [pltpu.VMEM((CH, DD), jnp.float32) for _ in range(KK)]
        + [pltpu.VMEM((CH,), jnp.int32) for _ in range(KK)]
        + [pltpu.SemaphoreType.DMA]
    ),
)


def _deg_body(dst_hbm, out_hbm, *sc):
    acc, zbuf, ones_v, dstag = sc[0], sc[1], sc[2], sc[3]
    didx = sc[4:4 + KK]
    c = lax.axis_index("c")
    s = lax.axis_index("s")
    base = c * HALF

    _zero_fill(zbuf, DD)
    for b in range(8):
        pltpu.sync_copy(zbuf, acc.at[pl.ds(s * 3128 + b * ZB_ROWS, ZB_ROWS)])
    ov = jnp.ones((16,), jnp.float32)

    def ob(i, carry):
        r = i // (DD // 16)
        g = i % (DD // 16)
        ones_v[r, pl.ds(g * 16, 16)] = ov
        return carry

    lax.fori_loop(0, CH * (DD // 16), ob, 0)
    plsc.subcore_barrier()

    e0 = s * EPT

    def step(i, carry):
        eoff = pl.multiple_of(e0 + i * SUP, 8)
        pltpu.sync_copy(dst_hbm.at[pl.ds(eoff, SUP)], dstag)
        for j in range(KK):
            _localize_dst(dstag, didx[j], j, base)
        for j in range(KK):
            pltpu.sync_copy(ones_v, acc.at[didx[j]], add=True)
        return carry

    lax.fori_loop(0, NSUP, step, 0)
    plsc.subcore_barrier()
    _writeback(acc, out_hbm, s, base)


_deg = pl.kernel(
    _deg_body,
    out_type=jax.ShapeDtypeStruct((NN, DD), jnp.float32),
    mesh=plsc.VectorSubcoreMesh(num_cores=NC, num_subcores=NS, **_MESH),
    compiler_params=pltpu.CompilerParams(use_tc_tiling_on_sc=False),
    scratch_types=(
        [
            pltpu.VMEM_SHARED((ACC_ROWS, DD), jnp.float32),
            pltpu.VMEM((ZB_ROWS, DD), jnp.float32),
            pltpu.VMEM((CH, DD), jnp.float32),
            pltpu.VMEM((SUP,), jnp.int32),
        ]
        + [pltpu.VMEM((CH,), jnp.int32) for _ in range(KK)]
    ),
)


def _gather3_body(feat_hbm, u_hbm, p_hbm, n_hbm, uo_hbm, po_hbm, no_hbm,
                  idxv, rows_v, sem):
    w = lax.axis_index("s") * NC + lax.axis_index("c")
    b0 = pl.multiple_of(w * (BB // (NC * NS)), 8)
    for ih, oh in ((u_hbm, uo_hbm), (p_hbm, po_hbm), (n_hbm, no_hbm)):
        pltpu.sync_copy(ih.at[pl.ds(b0, BB // (NC * NS))], idxv)
        pltpu.async_copy(feat_hbm.at[idxv], rows_v, sem).wait()
        pltpu.sync_copy(rows_v, oh.at[pl.ds(b0, BB // (NC * NS))])


_gather3 = pl.kernel(
    _gather3_body,
    out_type=(
        jax.ShapeDtypeStruct((BB, DD), jnp.float32),
        jax.ShapeDtypeStruct((BB, DD), jnp.float32),
        jax.ShapeDtypeStruct((BB, DD), jnp.float32),
    ),
    mesh=plsc.VectorSubcoreMesh(num_cores=NC, num_subcores=NS, **_MESH),
    compiler_params=pltpu.CompilerParams(use_tc_tiling_on_sc=False),
    scratch_types=[
        pltpu.VMEM((BB // (NC * NS),), jnp.int32),
        pltpu.VMEM((BB // (NC * NS), DD), jnp.float32),
        pltpu.SemaphoreType.DMA,
    ],
)


_ROWS = NN * DD // 128
_BLK = 1000
_GRID = _ROWS // _BLK
_spec = pl.BlockSpec((_BLK, 128), lambda i: (i, 0))
_fullshape = jax.ShapeDtypeStruct((_ROWS, 128), jnp.float32)


def _tc1_body(deg_ref, emb_ref, norm_ref, s0_ref):
    n = lax.rsqrt(jnp.maximum(deg_ref[...], 1.0))
    norm_ref[...] = n
    s0_ref[...] = emb_ref[...] * n


_tc1 = pl.pallas_call(
    _tc1_body,
    grid=(_GRID,),
    in_specs=[_spec, _spec],
    out_specs=[_spec, _spec],
    out_shape=[_fullshape, _fullshape],
)


def _tc2_body(a1_ref, norm_ref, h1_ref, s1_ref):
    n = norm_ref[...]
    h1 = a1_ref[...] * n
    h1_ref[...] = h1
    s1_ref[...] = h1 * n


_tc2 = pl.pallas_call(
    _tc2_body,
    grid=(_GRID,),
    in_specs=[_spec, _spec],
    out_specs=[_spec, _spec],
    out_shape=[_fullshape, _fullshape],
)


def _tc3_body(emb_ref, h1_ref, a2_ref, norm_ref, f_ref):
    f_ref[...] = (emb_ref[...] + h1_ref[...] + a2_ref[...] * norm_ref[...]) * (1.0 / 3.0)


_tc3 = pl.pallas_call(
    _tc3_body,
    grid=(_GRID,),
    in_specs=[_spec, _spec, _spec, _spec],
    out_specs=_spec,
    out_shape=_fullshape,
)


def _tc4_body(u_ref, p_ref, n_ref, ps_ref, ns_ref):
    u = u_ref[...]
    ps_ref[...] = jnp.sum(u * p_ref[...], axis=1, keepdims=True)
    ns_ref[...] = jnp.sum(u * n_ref[...], axis=1, keepdims=True)


_tc4 = pl.pallas_call(
    _tc4_body,
    out_shape=[
        jax.ShapeDtypeStruct((BB, 1), jnp.float32),
        jax.ShapeDtypeStruct((BB, 1), jnp.float32),
    ],
)


def kernel(embeddings, user, item_p, item_n, edge_index):
    src = edge_index[0]
    dst = edge_index[1]
    u_idx = user[:, 0]
    p_idx = item_p[:, 0] + NU
    n_idx = item_n[:, 0] + NU

    deg = _deg(dst)
    norm, s0 = _tc1(deg.reshape(_ROWS, 128), embeddings.reshape(_ROWS, 128))
    a1 = _prop(s0.reshape(NN, DD), src, dst)
    h1, s1 = _tc2(a1.reshape(_ROWS, 128), norm)
    a2 = _prop(s1.reshape(NN, DD), src, dst)
    feats = _tc3(embeddings.reshape(_ROWS, 128), h1, a2.reshape(_ROWS, 128), norm)
    ur, pr, nr = _gather3(feats.reshape(NN, DD), u_idx, p_idx, n_idx)
    p_score, n_score = _tc4(ur, pr, nr)
    return p_score, n_score


# speedup vs baseline: 5.3200x; 1.0062x over previous
import jax
import jax.numpy as jnp
from jax import lax
from jax.experimental import pallas as pl
from jax.experimental.pallas import tpu as pltpu, tpu_sc as plsc

NU = 50000
NN = 100000
DD = 32
EE = 1600000
BB = 4096

NC, NS = 2, 16
HALF = NN // NC
ACC_ROWS = 50048
ZB_ROWS = 391
CH = 80
KK = 5
NSUB = 10
SUP = CH * NSUB
EPT = EE // NS
NSUP = EPT // SUP
WB = 3128
WB_LAST = HALF - (NS - 1) * WB

_MESH = dict(core_axis_name="c", subcore_axis_name="s")


def _zero_fill(zbuf, width):
    zv = jnp.zeros((16,), jnp.float32)
    ngroups = width // 16

    def zb(i, carry):
        r = i // ngroups
        g = i % ngroups
        zbuf[r, pl.ds(g * 16, 16)] = zv
        return carry

    lax.fori_loop(0, ZB_ROWS * ngroups, zb, 0)


def _localize_dst(dstag, didx, j, base):
    for g in range(CH // 16):
        d = dstag[pl.ds(j * CH + g * 16, 16)]
        loc = d - base
        ok = (loc >= 0) & (loc < HALF)
        didx[pl.ds(g * 16, 16)] = jnp.where(ok, loc, HALF)


def _prop_body(feat_hbm, src_hbm, dst_hbm, out_hbm, *sc):
    acc, zbuf, sstag, dstag = sc[0], sc[1], sc[2], sc[3]
    rows = sc[4:4 + KK]
    didx = sc[4 + KK:4 + 2 * KK]
    sem = sc[4 + 2 * KK]
    c = lax.axis_index("c")
    s = lax.axis_index("s")
    base = c * HALF

    _zero_fill(zbuf, DD)
    for b in range(8):
        pltpu.sync_copy(zbuf, acc.at[pl.ds(s * 3128 + b * ZB_ROWS, ZB_ROWS)])
    plsc.subcore_barrier()

    e0 = s * EPT

    def step(i, carry):
        eoff = pl.multiple_of(e0 + i * SUP, 8)
        pltpu.sync_copy(src_hbm.at[pl.ds(eoff, SUP)], sstag)
        pltpu.sync_copy(dst_hbm.at[pl.ds(eoff, SUP)], dstag)
        descs = [None] * NSUB
        for j in range(KK):
            descs[j] = pltpu.async_copy(
                feat_hbm.at[sstag.at[pl.ds(j * CH, CH)]], rows[j], sem)
        for j in range(NSUB):
            descs[j].wait()
            _localize_dst(dstag, didx[j % KK], j, base)
            pltpu.sync_copy(rows[j % KK], acc.at[didx[j % KK]], add=True)
            if j + KK < NSUB:
                descs[j + KK] = pltpu.async_copy(
                    feat_hbm.at[sstag.at[pl.ds((j + KK) * CH, CH)]],
                    rows[(j + KK) % KK], sem)
        return carry

    lax.fori_loop(0, NSUP, step, 0)
    plsc.subcore_barrier()
    _writeback(acc, out_hbm, s, base)


def _writeback(acc, out_hbm, s, base):
    @pl.when(s < NS - 1)
    def _():
        off = pl.multiple_of(s * WB, 8)
        pltpu.sync_copy(acc.at[pl.ds(off, WB)],
                        out_hbm.at[pl.ds(pl.multiple_of(base + off, 8), WB)])

    @pl.when(s == NS - 1)
    def _():
        off = (NS - 1) * WB
        pltpu.sync_copy(acc.at[pl.ds(off, WB_LAST)],
                        out_hbm.at[pl.ds(pl.multiple_of(base + off, 8), WB_LAST)])


_prop = pl.kernel(
    _prop_body,
    out_type=jax.ShapeDtypeStruct((NN, DD), jnp.float32),
    mesh=plsc.VectorSubcoreMesh(num_cores=NC, num_subcores=NS, **_MESH),
    compiler_params=pltpu.CompilerParams(use_tc_tiling_on_sc=False),
    scratch_types=(
        [
            pltpu.VMEM_SHARED((ACC_ROWS, DD), jnp.float32),
            pltpu.VMEM((ZB_ROWS, DD), jnp.float32),
            pltpu.VMEM((SUP,), jnp.int32),
            pltpu.VMEM((SUP,), jnp.int32),
        ]
        + [pltpu.VMEM((CH, DD), jnp.float32) for _ in range(KK)]
        + [pltpu.VMEM((CH,), jnp.int32) for _ in range(KK)]
        + [pltpu.SemaphoreType.DMA]
    ),
)


def _deg_body(dst_hbm, out_hbm, *sc):
    acc, zbuf, ones_v, dstag = sc[0], sc[1], sc[2], sc[3]
    didx = sc[4:4 + KK]
    c = lax.axis_index("c")
    s = lax.axis_index("s")
    base = c * HALF

    _zero_fill(zbuf, DD)
    for b in range(8):
        pltpu.sync_copy(zbuf, acc.at[pl.ds(s * 3128 + b * ZB_ROWS, ZB_ROWS)])
    ov = jnp.ones((16,), jnp.float32)

    def ob(i, carry):
        r = i // (DD // 16)
        g = i % (DD // 16)
        ones_v[r, pl.ds(g * 16, 16)] = ov
        return carry

    lax.fori_loop(0, CH * (DD // 16), ob, 0)
    plsc.subcore_barrier()

    e0 = s * EPT

    def step(i, carry):
        eoff = pl.multiple_of(e0 + i * SUP, 8)
        pltpu.sync_copy(dst_hbm.at[pl.ds(eoff, SUP)], dstag)
        for j in range(NSUB):
            _localize_dst(dstag, didx[j % KK], j, base)
            pltpu.sync_copy(ones_v, acc.at[didx[j % KK]], add=True)
        return carry

    lax.fori_loop(0, NSUP, step, 0)
    plsc.subcore_barrier()
    _writeback(acc, out_hbm, s, base)


_deg = pl.kernel(
    _deg_body,
    out_type=jax.ShapeDtypeStruct((NN, DD), jnp.float32),
    mesh=plsc.VectorSubcoreMesh(num_cores=NC, num_subcores=NS, **_MESH),
    compiler_params=pltpu.CompilerParams(use_tc_tiling_on_sc=False),
    scratch_types=(
        [
            pltpu.VMEM_SHARED((ACC_ROWS, DD), jnp.float32),
            pltpu.VMEM((ZB_ROWS, DD), jnp.float32),
            pltpu.VMEM((CH, DD), jnp.float32),
            pltpu.VMEM((SUP,), jnp.int32),
        ]
        + [pltpu.VMEM((CH,), jnp.int32) for _ in range(KK)]
    ),
)


def _gather3_body(feat_hbm, u_hbm, p_hbm, n_hbm, uo_hbm, po_hbm, no_hbm,
                  idxv, rows_v, sem):
    w = lax.axis_index("s") * NC + lax.axis_index("c")
    b0 = pl.multiple_of(w * (BB // (NC * NS)), 8)
    for ih, oh in ((u_hbm, uo_hbm), (p_hbm, po_hbm), (n_hbm, no_hbm)):
        pltpu.sync_copy(ih.at[pl.ds(b0, BB // (NC * NS))], idxv)
        pltpu.async_copy(feat_hbm.at[idxv], rows_v, sem).wait()
        pltpu.sync_copy(rows_v, oh.at[pl.ds(b0, BB // (NC * NS))])


_gather3 = pl.kernel(
    _gather3_body,
    out_type=(
        jax.ShapeDtypeStruct((BB, DD), jnp.float32),
        jax.ShapeDtypeStruct((BB, DD), jnp.float32),
        jax.ShapeDtypeStruct((BB, DD), jnp.float32),
    ),
    mesh=plsc.VectorSubcoreMesh(num_cores=NC, num_subcores=NS, **_MESH),
    compiler_params=pltpu.CompilerParams(use_tc_tiling_on_sc=False),
    scratch_types=[
        pltpu.VMEM((BB // (NC * NS),), jnp.int32),
        pltpu.VMEM((BB // (NC * NS), DD), jnp.float32),
        pltpu.SemaphoreType.DMA,
    ],
)


_ROWS = NN * DD // 128
_BLK = 1000
_GRID = _ROWS // _BLK
_spec = pl.BlockSpec((_BLK, 128), lambda i: (i, 0))
_fullshape = jax.ShapeDtypeStruct((_ROWS, 128), jnp.float32)


def _tc1_body(deg_ref, emb_ref, norm_ref, s0_ref):
    n = lax.rsqrt(jnp.maximum(deg_ref[...], 1.0))
    norm_ref[...] = n
    s0_ref[...] = emb_ref[...] * n


_tc1 = pl.pallas_call(
    _tc1_body,
    grid=(_GRID,),
    in_specs=[_spec, _spec],
    out_specs=[_spec, _spec],
    out_shape=[_fullshape, _fullshape],
)


def _tc2_body(a1_ref, norm_ref, h1_ref, s1_ref):
    n = norm_ref[...]
    h1 = a1_ref[...] * n
    h1_ref[...] = h1
    s1_ref[...] = h1 * n


_tc2 = pl.pallas_call(
    _tc2_body,
    grid=(_GRID,),
    in_specs=[_spec, _spec],
    out_specs=[_spec, _spec],
    out_shape=[_fullshape, _fullshape],
)


def _tc3_body(emb_ref, h1_ref, a2_ref, norm_ref, f_ref):
    f_ref[...] = (emb_ref[...] + h1_ref[...] + a2_ref[...] * norm_ref[...]) * (1.0 / 3.0)


_tc3 = pl.pallas_call(
    _tc3_body,
    grid=(_GRID,),
    in_specs=[_spec, _spec, _spec, _spec],
    out_specs=_spec,
    out_shape=_fullshape,
)


def _tc4_body(u_ref, p_ref, n_ref, ps_ref, ns_ref):
    u = u_ref[...]
    ps_ref[...] = jnp.sum(u * p_ref[...], axis=1, keepdims=True)
    ns_ref[...] = jnp.sum(u * n_ref[...], axis=1, keepdims=True)


_tc4 = pl.pallas_call(
    _tc4_body,
    out_shape=[
        jax.ShapeDtypeStruct((BB, 1), jnp.float32),
        jax.ShapeDtypeStruct((BB, 1), jnp.float32),
    ],
)


def kernel(embeddings, user, item_p, item_n, edge_index):
    src = edge_index[0]
    dst = edge_index[1]
    u_idx = user[:, 0]
    p_idx = item_p[:, 0] + NU
    n_idx = item_n[:, 0] + NU

    deg = _deg(dst)
    norm, s0 = _tc1(deg.reshape(_ROWS, 128), embeddings.reshape(_ROWS, 128))
    a1 = _prop(s0.reshape(NN, DD), src, dst)
    h1, s1 = _tc2(a1.reshape(_ROWS, 128), norm)
    a2 = _prop(s1.reshape(NN, DD), src, dst)
    feats = _tc3(embeddings.reshape(_ROWS, 128), h1, a2.reshape(_ROWS, 128), norm)
    ur, pr, nr = _gather3(feats.reshape(NN, DD), u_idx, p_idx, n_idx)
    p_score, n_score = _tc4(ur, pr, nr)
    return p_score, n_score


# speedup vs baseline: 5.3215x; 1.0003x over previous
import jax
import jax.numpy as jnp
from jax import lax
from jax.experimental import pallas as pl
from jax.experimental.pallas import tpu as pltpu, tpu_sc as plsc

NU = 50000
NN = 100000
DD = 32
EE = 1600000
BB = 4096

NC, NS = 2, 16
HALF = NN // NC
ACC_ROWS = 50048
ZB_ROWS = 391
CH = 80
KK = 5
NSUB = 10
SUP = CH * NSUB
EPT = EE // NS
NSUP = EPT // SUP
WB = 3128
WB_LAST = HALF - (NS - 1) * WB

_MESH = dict(core_axis_name="c", subcore_axis_name="s")


def _zero_fill(zbuf, width):
    zv = jnp.zeros((16,), jnp.float32)
    ngroups = width // 16

    def zb(i, carry):
        r = i // ngroups
        g = i % ngroups
        zbuf[r, pl.ds(g * 16, 16)] = zv
        return carry

    lax.fori_loop(0, ZB_ROWS * ngroups, zb, 0)


def _localize_dst(dstag, didx, j, base):
    for g in range(CH // 16):
        d = dstag[pl.ds(j * CH + g * 16, 16)]
        loc = d - base
        ok = (loc >= 0) & (loc < HALF)
        didx[pl.ds(g * 16, 16)] = jnp.where(ok, loc, HALF)


def _prop_body(feat_hbm, src_hbm, dst_hbm, out_hbm, *sc):
    acc, zbuf, sstag, dstag = sc[0], sc[1], sc[2], sc[3]
    rows = sc[4:4 + KK]
    didx = sc[4 + KK:4 + 2 * KK]
    sem = sc[4 + 2 * KK]
    c = lax.axis_index("c")
    s = lax.axis_index("s")
    base = c * HALF

    _zero_fill(zbuf, DD)
    for b in range(8):
        pltpu.sync_copy(zbuf, acc.at[pl.ds(s * 3128 + b * ZB_ROWS, ZB_ROWS)])
    plsc.subcore_barrier()

    e0 = s * EPT

    def step(i, carry):
        eoff = pl.multiple_of(e0 + i * SUP, 8)
        pltpu.sync_copy(src_hbm.at[pl.ds(eoff, SUP)], sstag)
        pltpu.sync_copy(dst_hbm.at[pl.ds(eoff, SUP)], dstag)
        descs = [None] * NSUB
        for j in range(KK):
            descs[j] = pltpu.async_copy(
                feat_hbm.at[sstag.at[pl.ds(j * CH, CH)]], rows[j], sem)
        for j in range(NSUB):
            descs[j].wait()
            _localize_dst(dstag, didx[j % KK], j, base)
            pltpu.sync_copy(rows[j % KK], acc.at[didx[j % KK]], add=True)
            if j + KK < NSUB:
                descs[j + KK] = pltpu.async_copy(
                    feat_hbm.at[sstag.at[pl.ds((j + KK) * CH, CH)]],
                    rows[(j + KK) % KK], sem)
        return carry

    lax.fori_loop(0, NSUP, step, 0)
    plsc.subcore_barrier()
    _writeback(acc, out_hbm, s, base)


def _writeback(acc, out_hbm, s, base):
    @pl.when(s < NS - 1)
    def _():
        off = pl.multiple_of(s * WB, 8)
        pltpu.sync_copy(acc.at[pl.ds(off, WB)],
                        out_hbm.at[pl.ds(pl.multiple_of(base + off, 8), WB)])

    @pl.when(s == NS - 1)
    def _():
        off = (NS - 1) * WB
        pltpu.sync_copy(acc.at[pl.ds(off, WB_LAST)],
                        out_hbm.at[pl.ds(pl.multiple_of(base + off, 8), WB_LAST)])


_prop = pl.kernel(
    _prop_body,
    out_type=jax.ShapeDtypeStruct((NN, DD), jnp.float32),
    mesh=plsc.VectorSubcoreMesh(num_cores=NC, num_subcores=NS, **_MESH),
    compiler_params=pltpu.CompilerParams(use_tc_tiling_on_sc=False),
    scratch_types=(
        [
            pltpu.VMEM_SHARED((ACC_ROWS, DD), jnp.float32),
            pltpu.VMEM((ZB_ROWS, DD), jnp.float32),
            pltpu.VMEM((SUP,), jnp.int32),
            pltpu.VMEM((SUP,), jnp.int32),
        ]
        + [pltpu.VMEM((CH, DD), jnp.float32) for _ in range(KK)]
        + [pltpu.VMEM((CH,), jnp.int32) for _ in range(KK)]
        + [pltpu.SemaphoreType.DMA]
    ),
)


def _deg_body(dst_hbm, out_hbm, *sc):
    acc, zbuf, ones_v, dstag = sc[0], sc[1], sc[2], sc[3]
    didx = sc[4:4 + NSUB]
    sem = sc[4 + NSUB]
    c = lax.axis_index("c")
    s = lax.axis_index("s")
    base = c * HALF

    _zero_fill(zbuf, DD)
    for b in range(8):
        pltpu.sync_copy(zbuf, acc.at[pl.ds(s * 3128 + b * ZB_ROWS, ZB_ROWS)])
    ov = jnp.ones((16,), jnp.float32)

    def ob(i, carry):
        r = i // (DD // 16)
        g = i % (DD // 16)
        ones_v[r, pl.ds(g * 16, 16)] = ov
        return carry

    lax.fori_loop(0, CH * (DD // 16), ob, 0)
    plsc.subcore_barrier()

    e0 = s * EPT

    def step(i, carry):
        eoff = pl.multiple_of(e0 + i * SUP, 8)
        pltpu.sync_copy(dst_hbm.at[pl.ds(eoff, SUP)], dstag)
        for j in range(NSUB):
            _localize_dst(dstag, didx[j], j, base)
        sds = [pltpu.async_copy(ones_v, acc.at[didx[j]], sem, add=True)
               for j in range(NSUB)]
        for d in sds:
            d.wait()
        return carry

    lax.fori_loop(0, NSUP, step, 0)
    plsc.subcore_barrier()
    _writeback(acc, out_hbm, s, base)


_deg = pl.kernel(
    _deg_body,
    out_type=jax.ShapeDtypeStruct((NN, DD), jnp.float32),
    mesh=plsc.VectorSubcoreMesh(num_cores=NC, num_subcores=NS, **_MESH),
    compiler_params=pltpu.CompilerParams(use_tc_tiling_on_sc=False),
    scratch_types=(
        [
            pltpu.VMEM_SHARED((ACC_ROWS, DD), jnp.float32),
            pltpu.VMEM((ZB_ROWS, DD), jnp.float32),
            pltpu.VMEM((CH, DD), jnp.float32),
            pltpu.VMEM((SUP,), jnp.int32),
        ]
        + [pltpu.VMEM((CH,), jnp.int32) for _ in range(NSUB)]
        + [pltpu.SemaphoreType.DMA]
    ),
)


def _gather3_body(feat_hbm, u_hbm, p_hbm, n_hbm, uo_hbm, po_hbm, no_hbm,
                  idxv, rows_v, sem):
    w = lax.axis_index("s") * NC + lax.axis_index("c")
    b0 = pl.multiple_of(w * (BB // (NC * NS)), 8)
    for ih, oh in ((u_hbm, uo_hbm), (p_hbm, po_hbm), (n_hbm, no_hbm)):
        pltpu.sync_copy(ih.at[pl.ds(b0, BB // (NC * NS))], idxv)
        pltpu.async_copy(feat_hbm.at[idxv], rows_v, sem).wait()
        pltpu.sync_copy(rows_v, oh.at[pl.ds(b0, BB // (NC * NS))])


_gather3 = pl.kernel(
    _gather3_body,
    out_type=(
        jax.ShapeDtypeStruct((BB, DD), jnp.float32),
        jax.ShapeDtypeStruct((BB, DD), jnp.float32),
        jax.ShapeDtypeStruct((BB, DD), jnp.float32),
    ),
    mesh=plsc.VectorSubcoreMesh(num_cores=NC, num_subcores=NS, **_MESH),
    compiler_params=pltpu.CompilerParams(use_tc_tiling_on_sc=False),
    scratch_types=[
        pltpu.VMEM((BB // (NC * NS),), jnp.int32),
        pltpu.VMEM((BB // (NC * NS), DD), jnp.float32),
        pltpu.SemaphoreType.DMA,
    ],
)


_ROWS = NN * DD // 128
_BLK = 1000
_GRID = _ROWS // _BLK
_spec = pl.BlockSpec((_BLK, 128), lambda i: (i, 0))
_fullshape = jax.ShapeDtypeStruct((_ROWS, 128), jnp.float32)


def _tc1_body(deg_ref, emb_ref, norm_ref, s0_ref):
    n = lax.rsqrt(jnp.maximum(deg_ref[...], 1.0))
    norm_ref[...] = n
    s0_ref[...] = emb_ref[...] * n


_tc1 = pl.pallas_call(
    _tc1_body,
    grid=(_GRID,),
    in_specs=[_spec, _spec],
    out_specs=[_spec, _spec],
    out_shape=[_fullshape, _fullshape],
)


def _tc2_body(a1_ref, norm_ref, h1_ref, s1_ref):
    n = norm_ref[...]
    h1 = a1_ref[...] * n
    h1_ref[...] = h1
    s1_ref[...] = h1 * n


_tc2 = pl.pallas_call(
    _tc2_body,
    grid=(_GRID,),
    in_specs=[_spec, _spec],
    out_specs=[_spec, _spec],
    out_shape=[_fullshape, _fullshape],
)


def _tc3_body(emb_ref, h1_ref, a2_ref, norm_ref, f_ref):
    f_ref[...] = (emb_ref[...] + h1_ref[...] + a2_ref[...] * norm_ref[...]) * (1.0 / 3.0)


_tc3 = pl.pallas_call(
    _tc3_body,
    grid=(_GRID,),
    in_specs=[_spec, _spec, _spec, _spec],
    out_specs=_spec,
    out_shape=_fullshape,
)


def _tc4_body(u_ref, p_ref, n_ref, ps_ref, ns_ref):
    u = u_ref[...]
    ps_ref[...] = jnp.sum(u * p_ref[...], axis=1, keepdims=True)
    ns_ref[...] = jnp.sum(u * n_ref[...], axis=1, keepdims=True)


_tc4 = pl.pallas_call(
    _tc4_body,
    out_shape=[
        jax.ShapeDtypeStruct((BB, 1), jnp.float32),
        jax.ShapeDtypeStruct((BB, 1), jnp.float32),
    ],
)


def kernel(embeddings, user, item_p, item_n, edge_index):
    src = edge_index[0]
    dst = edge_index[1]
    u_idx = user[:, 0]
    p_idx = item_p[:, 0] + NU
    n_idx = item_n[:, 0] + NU

    deg = _deg(dst)
    norm, s0 = _tc1(deg.reshape(_ROWS, 128), embeddings.reshape(_ROWS, 128))
    a1 = _prop(s0.reshape(NN, DD), src, dst)
    h1, s1 = _tc2(a1.reshape(_ROWS, 128), norm)
    a2 = _prop(s1.reshape(NN, DD), src, dst)
    feats = _tc3(embeddings.reshape(_ROWS, 128), h1, a2.reshape(_ROWS, 128), norm)
    ur, pr, nr = _gather3(feats.reshape(NN, DD), u_idx, p_idx, n_idx)
    p_score, n_score = _tc4(ur, pr, nr)
    return p_score, n_score
